# baseline (speedup 1.0000x reference)
import functools

import jax
import jax.numpy as jnp
from jax import lax
from jax.experimental import pallas as pl
from jax.experimental.pallas import tpu as pltpu
from jax.experimental.pallas import tpu_sc as plsc

N = 10000
D = 128
HD = D // 2
NC = 2
NS = 16
C = 128
CPT = 160
EPT = C * CPT
EPAD = EPT * NS
NB = 4
RPS = 624
TAIL0 = RPS * NS
TAILN = N - TAIL0


def _sc_spmm(z_t, srcp, dstp, w1, a16):
    mesh = plsc.VectorSubcoreMesh(core_axis_name="c", subcore_axis_name="s",
                                  num_cores=NC, num_subcores=NS)

    @functools.partial(
        pl.kernel,
        out_type=jax.ShapeDtypeStruct((NC, N, HD), jnp.float32),
        mesh=mesh,
        compiler_params=pltpu.CompilerParams(use_tc_tiling_on_sc=False),
        scratch_types=dict(
            srcp_v=pltpu.VMEM((CPT, C // 2), jnp.int32),
            dstp_v=pltpu.VMEM((CPT, C // 2), jnp.int32),
            w_v=pltpu.VMEM((CPT, C), jnp.float32),
            a_v=pltpu.VMEM((16,), jnp.float32),
            isrc=[pltpu.VMEM((C,), jnp.int32) for _ in range(NB)],
            idst=[pltpu.VMEM((C,), jnp.int32) for _ in range(NB)],
            rows=[pltpu.VMEM((C, HD), jnp.float32) for _ in range(NB)],
            gsem=[pltpu.SemaphoreType.DMA for _ in range(NB)],
            ssem=[pltpu.SemaphoreType.DMA for _ in range(NB)],
            acc=pltpu.VMEM_SHARED((N, HD), jnp.float32),
        ),
    )
    def k(z_hbm, src_hbm, dst_hbm, w_hbm, a_hbm, out_hbm,
          srcp_v, dstp_v, w_v, a_v, isrc, idst, rows, gsem, ssem, acc):
        cid = lax.axis_index("c")
        sid = lax.axis_index("s")

        def zrow(e, carry):
            for j in range(HD // 16):
                rows[0][e, pl.ds(16 * j, 16)] = jnp.zeros((16,), jnp.float32)
            return carry
        lax.fori_loop(0, C, zrow, 0)
        for off in range(0, RPS, C):
            sz = min(C, RPS - off)
            pltpu.sync_copy(rows[0].at[pl.ds(0, sz)],
                            acc.at[pl.ds(sid * RPS + off, sz)])
        @pl.when(sid == NS - 1)
        def _zero_tail():
            pltpu.sync_copy(rows[0].at[pl.ds(0, TAILN)],
                            acc.at[pl.ds(TAIL0, TAILN)])
        plsc.subcore_barrier()

        pltpu.sync_copy(src_hbm.at[pl.ds(sid * CPT, CPT)], srcp_v)
        pltpu.sync_copy(dst_hbm.at[pl.ds(sid * CPT, CPT)], dstp_v)
        pltpu.sync_copy(w_hbm.at[pl.ds(sid * CPT, CPT)], w_v)
        pltpu.sync_copy(a_hbm, a_v)
        av16 = a_v[...]
        def wscale(i, carry):
            for j in range(C // 16):
                w_v[i, pl.ds(16 * j, 16)] = w_v[i, pl.ds(16 * j, 16)] * av16
            return carry
        lax.fori_loop(0, CPT, wscale, 0)

        def build_idx(buf, packed_ref, g):
            for i in range(C // 32):
                wvec = packed_ref[g, pl.ds(16 * i, 16)]
                buf[pl.ds(32 * i, 16)] = jnp.bitwise_and(wvec, 0xFFFF)
                buf[pl.ds(32 * i + 16, 16)] = jnp.right_shift(wvec, 16)

        def start_gather(b, g):
            build_idx(isrc[b], srcp_v, g)
            build_idx(idst[b], dstp_v, g)
            pltpu.async_copy(z_hbm.at[cid].at[isrc[b]], rows[b], gsem[b])

        def wait_gather(b):
            pltpu.make_async_copy(z_hbm.at[cid].at[isrc[b]], rows[b],
                                  gsem[b]).wait()

        def start_scatter(b):
            pltpu.async_copy(rows[b], acc.at[idst[b]], ssem[b], add=True)

        def wait_scatter(b):
            pltpu.make_async_copy(rows[b], acc.at[idst[b]], ssem[b]).wait()

        for b0 in range(NB - 1):
            start_gather(b0, b0)

        def outer(t, carry):
            for b in range(NB):
                g = t * NB + b
                wait_gather(b)
                bp = (b + NB - 1) % NB
                if b == 0:
                    @pl.when(t > 0)
                    def _drain0():
                        wait_scatter(bp)
                else:
                    wait_scatter(bp)
                @pl.when(g + NB - 1 < CPT)
                def _next():
                    start_gather(bp, g + NB - 1)
                def scale(eg, c2):
                    wv16 = w_v[g, pl.ds(16 * eg, 16)]
                    for l in range(16):
                        e = 16 * eg + l
                        for j in range(HD // 16):
                            rows[b][e, pl.ds(16 * j, 16)] = (
                                rows[b][e, pl.ds(16 * j, 16)] * wv16[l])
                    return c2
                lax.fori_loop(0, C // 16, scale, 0)
                start_scatter(b)
            return carry
        lax.fori_loop(0, CPT // NB, outer, 0)
        wait_scatter((CPT - 1) % NB)
        plsc.subcore_barrier()

        for off in range(0, RPS, C):
            sz = min(C, RPS - off)
            pltpu.sync_copy(acc.at[pl.ds(sid * RPS + off, sz)],
                            rows[0].at[pl.ds(0, sz)])
            pltpu.sync_copy(rows[0].at[pl.ds(0, sz)],
                            out_hbm.at[cid].at[pl.ds(sid * RPS + off, sz)])
        @pl.when(sid == NS - 1)
        def _write_tail():
            pltpu.sync_copy(acc.at[pl.ds(TAIL0, TAILN)],
                            rows[1].at[pl.ds(0, TAILN)])
            pltpu.sync_copy(rows[1].at[pl.ds(0, TAILN)],
                            out_hbm.at[cid].at[pl.ds(TAIL0, TAILN)])

    return k(z_t, srcp, dstp, w1, a16)


def _pack16(x2):
    g = x2.reshape(-1, 2, 16)
    return (g[:, 0, :] | (g[:, 1, :] << 16)).reshape(EPAD // C, C // 2)


def kernel(Z, edge_index, edge_weight, alpha):
    src = edge_index[0].astype(jnp.int32)
    dst = edge_index[1].astype(jnp.int32)
    w = edge_weight.astype(jnp.float32)
    pad = EPAD - src.shape[0]
    srcp = _pack16(jnp.pad(src, (0, pad)).reshape(EPAD // C, C))
    dstp = _pack16(jnp.pad(dst, (0, pad)).reshape(EPAD // C, C))
    w1 = jnp.pad(w, (0, pad)).reshape(EPAD // C, C)
    a16 = jnp.broadcast_to(alpha.astype(jnp.float32)[None], (16,))
    z_t = jnp.swapaxes(Z.reshape(N, NC, HD), 0, 1)
    out_t = _sc_spmm(z_t, srcp, dstp, w1, a16)
    return jnp.swapaxes(out_t, 0, 1).reshape(N, D)


# speedup vs baseline: 1.0518x; 1.0518x over previous
import functools

import jax
import jax.numpy as jnp
from jax import lax
from jax.experimental import pallas as pl
from jax.experimental.pallas import tpu as pltpu
from jax.experimental.pallas import tpu_sc as plsc

N = 10000
D = 128
HD = D // 2
NC = 2
NS = 16
C = 128
CPT = 160
EPT = C * CPT
EPAD = EPT * NS
NB = 4
RPS = 624
TAIL0 = RPS * NS
TAILN = N - TAIL0


def _sc_spmm(z_t, srcp, dstp, w1, a16):
    mesh = plsc.VectorSubcoreMesh(core_axis_name="c", subcore_axis_name="s",
                                  num_cores=NC, num_subcores=NS)

    @functools.partial(
        pl.kernel,
        out_type=jax.ShapeDtypeStruct((NC, N, HD), jnp.float32),
        mesh=mesh,
        compiler_params=pltpu.CompilerParams(use_tc_tiling_on_sc=False),
        scratch_types=dict(
            srcp_v=pltpu.VMEM((CPT, C // 2), jnp.int32),
            dstp_v=pltpu.VMEM((CPT, C // 2), jnp.int32),
            w_v=pltpu.VMEM((CPT, C), jnp.float32),
            a_v=pltpu.VMEM((16,), jnp.float32),
            isrc=[pltpu.VMEM((C,), jnp.int32) for _ in range(NB)],
            idst=[pltpu.VMEM((C,), jnp.int32) for _ in range(NB)],
            rows=[pltpu.VMEM((C, HD), jnp.float32) for _ in range(NB)],
            gsem=[pltpu.SemaphoreType.DMA for _ in range(NB)],
            ssem=[pltpu.SemaphoreType.DMA for _ in range(NB)],
            acc=pltpu.VMEM_SHARED((N, HD), jnp.float32),
        ),
    )
    def k(z_hbm, src_hbm, dst_hbm, w_hbm, a_hbm, out_hbm,
          srcp_v, dstp_v, w_v, a_v, isrc, idst, rows, gsem, ssem, acc):
        cid = lax.axis_index("c")
        sid = lax.axis_index("s")

        def zrow(e, carry):
            for j in range(HD // 16):
                rows[0][e, pl.ds(16 * j, 16)] = jnp.zeros((16,), jnp.float32)
            return carry
        lax.fori_loop(0, C, zrow, 0)
        for off in range(0, RPS, C):
            sz = min(C, RPS - off)
            pltpu.sync_copy(rows[0].at[pl.ds(0, sz)],
                            acc.at[pl.ds(sid * RPS + off, sz)])
        @pl.when(sid == NS - 1)
        def _zero_tail():
            pltpu.sync_copy(rows[0].at[pl.ds(0, TAILN)],
                            acc.at[pl.ds(TAIL0, TAILN)])
        plsc.subcore_barrier()

        pltpu.sync_copy(src_hbm.at[pl.ds(sid * CPT, CPT)], srcp_v)
        pltpu.sync_copy(dst_hbm.at[pl.ds(sid * CPT, CPT)], dstp_v)
        pltpu.sync_copy(w_hbm.at[pl.ds(sid * CPT, CPT)], w_v)
        pltpu.sync_copy(a_hbm, a_v)
        av16 = a_v[...]
        def wscale(i, carry):
            for j in range(C // 16):
                w_v[i, pl.ds(16 * j, 16)] = w_v[i, pl.ds(16 * j, 16)] * av16
            return carry
        lax.fori_loop(0, CPT, wscale, 0)

        def build_idx(buf, packed_ref, g):
            for i in range(C // 32):
                wvec = packed_ref[g, pl.ds(16 * i, 16)]
                buf[pl.ds(32 * i, 16)] = jnp.bitwise_and(wvec, 0xFFFF)
                buf[pl.ds(32 * i + 16, 16)] = jnp.right_shift(wvec, 16)

        def start_gather(b, g):
            build_idx(isrc[b], srcp_v, g)
            build_idx(idst[b], dstp_v, g)
            pltpu.async_copy(z_hbm.at[cid].at[isrc[b]], rows[b], gsem[b])

        def wait_gather(b):
            pltpu.make_async_copy(z_hbm.at[cid].at[isrc[b]], rows[b],
                                  gsem[b]).wait()

        def start_scatter(b):
            pltpu.async_copy(rows[b], acc.at[idst[b]], ssem[b], add=True)

        def wait_scatter(b):
            pltpu.make_async_copy(rows[b], acc.at[idst[b]], ssem[b]).wait()

        for b0 in range(NB - 2):
            start_gather(b0, b0)

        def outer(t, carry):
            for b in range(NB):
                g = t * NB + b
                wait_gather(b)
                bp = (b + 2) % NB
                if b < 2:
                    @pl.when(t > 0)
                    def _drain0():
                        wait_scatter(bp)
                else:
                    wait_scatter(bp)
                @pl.when(g + 2 < CPT)
                def _next():
                    start_gather(bp, g + 2)
                def scale(eg, c2):
                    wv16 = w_v[g, pl.ds(16 * eg, 16)]
                    for l in range(16):
                        e = 16 * eg + l
                        for j in range(HD // 16):
                            rows[b][e, pl.ds(16 * j, 16)] = (
                                rows[b][e, pl.ds(16 * j, 16)] * wv16[l])
                    return c2
                lax.fori_loop(0, C // 16, scale, 0)
                start_scatter(b)
            return carry
        lax.fori_loop(0, CPT // NB, outer, 0)
        wait_scatter((CPT - 2) % NB)
        wait_scatter((CPT - 1) % NB)
        plsc.subcore_barrier()

        for off in range(0, RPS, C):
            sz = min(C, RPS - off)
            pltpu.sync_copy(acc.at[pl.ds(sid * RPS + off, sz)],
                            rows[0].at[pl.ds(0, sz)])
            pltpu.sync_copy(rows[0].at[pl.ds(0, sz)],
                            out_hbm.at[cid].at[pl.ds(sid * RPS + off, sz)])
        @pl.when(sid == NS - 1)
        def _write_tail():
            pltpu.sync_copy(acc.at[pl.ds(TAIL0, TAILN)],
                            rows[1].at[pl.ds(0, TAILN)])
            pltpu.sync_copy(rows[1].at[pl.ds(0, TAILN)],
                            out_hbm.at[cid].at[pl.ds(TAIL0, TAILN)])

    return k(z_t, srcp, dstp, w1, a16)


def _pack16(x2):
    g = x2.reshape(-1, 2, 16)
    return (g[:, 0, :] | (g[:, 1, :] << 16)).reshape(EPAD // C, C // 2)


def kernel(Z, edge_index, edge_weight, alpha):
    src = edge_index[0].astype(jnp.int32)
    dst = edge_index[1].astype(jnp.int32)
    w = edge_weight.astype(jnp.float32)
    pad = EPAD - src.shape[0]
    srcp = _pack16(jnp.pad(src, (0, pad)).reshape(EPAD // C, C))
    dstp = _pack16(jnp.pad(dst, (0, pad)).reshape(EPAD // C, C))
    w1 = jnp.pad(w, (0, pad)).reshape(EPAD // C, C)
    a16 = jnp.broadcast_to(alpha.astype(jnp.float32)[None], (16,))
    z_t = jnp.swapaxes(Z.reshape(N, NC, HD), 0, 1)
    out_t = _sc_spmm(z_t, srcp, dstp, w1, a16)
    return jnp.swapaxes(out_t, 0, 1).reshape(N, D)


# speedup vs baseline: 1.9005x; 1.8070x over previous
import functools

import jax
import jax.numpy as jnp
from jax import lax
from jax.experimental import pallas as pl
from jax.experimental.pallas import tpu as pltpu
from jax.experimental.pallas import tpu_sc as plsc

N = 10000
D = 128
HD = D // 2
NC = 2
NS = 16
C = 128
CPT = 159
EPT = C * CPT
EPAD = EPT * NS
NB = 3
RPS = 624
TAIL0 = RPS * NS
TAILN = N - TAIL0


def _sc_spmm(z3, src2, dst2, w1, a8):
    mesh = plsc.VectorSubcoreMesh(core_axis_name="c", subcore_axis_name="s",
                                  num_cores=NC, num_subcores=NS)

    @functools.partial(
        pl.kernel,
        out_type=jax.ShapeDtypeStruct((NC, N, HD), jnp.float32),
        mesh=mesh,
        compiler_params=pltpu.CompilerParams(use_tc_tiling_on_sc=False,
                                             needs_layout_passes=False),
        scratch_types=dict(
            src_v=pltpu.VMEM((CPT, C), jnp.int32),
            dst_v=pltpu.VMEM((CPT, C), jnp.int32),
            wch=[pltpu.VMEM((C,), jnp.float32) for _ in range(NB)],
            a_v=pltpu.VMEM((16,), jnp.float32),
            rows_g=[pltpu.VMEM((C, HD), jnp.bfloat16) for _ in range(NB)],
            rows_s=[pltpu.VMEM((C, HD), jnp.float32) for _ in range(NB)],
            gsem=[pltpu.SemaphoreType.DMA for _ in range(NB)],
            ssem=[pltpu.SemaphoreType.DMA for _ in range(NB)],
            wsem=[pltpu.SemaphoreType.DMA for _ in range(NB)],
            acc=pltpu.VMEM_SHARED((N, HD), jnp.float32),
        ),
    )
    def k(z_hbm, src_hbm, dst_hbm, w_hbm, a_hbm, out_hbm,
          src_v, dst_v, wch, a_v, rows_g, rows_s, gsem, ssem, wsem, acc):
        cid = lax.axis_index("c")
        sid = lax.axis_index("s")

        def zrow(e, carry):
            for j in range(HD // 16):
                rows_s[0][e, pl.ds(16 * j, 16)] = jnp.zeros((16,), jnp.float32)
            return carry
        lax.fori_loop(0, C, zrow, 0)
        for off in range(0, RPS, C):
            sz = min(C, RPS - off)
            pltpu.sync_copy(rows_s[0].at[pl.ds(0, sz)],
                            acc.at[pl.ds(sid * RPS + off, sz)])
        @pl.when(sid == NS - 1)
        def _zero_tail():
            pltpu.sync_copy(rows_s[0].at[pl.ds(0, TAILN)],
                            acc.at[pl.ds(TAIL0, TAILN)])
        plsc.subcore_barrier()

        pltpu.sync_copy(src_hbm.at[pl.ds(sid * CPT, CPT)], src_v)
        pltpu.sync_copy(dst_hbm.at[pl.ds(sid * CPT, CPT)], dst_v)
        pltpu.sync_copy(a_hbm, a_v)
        av16 = a_v[...]

        def start_gather(b, g):
            pltpu.async_copy(w_hbm.at[sid * CPT + g], wch[b], wsem[b])
            pltpu.async_copy(z_hbm.at[cid].at[src_v.at[g]], rows_g[b],
                             gsem[b])

        def wait_gather(b, g):
            pltpu.make_async_copy(w_hbm.at[sid * CPT + g], wch[b],
                                  wsem[b]).wait()
            pltpu.make_async_copy(z_hbm.at[cid].at[src_v.at[g]], rows_g[b],
                                  gsem[b]).wait()

        def start_scatter(b, g):
            pltpu.async_copy(rows_s[b], acc.at[dst_v.at[g]], ssem[b],
                             add=True)

        def wait_scatter(b, g):
            pltpu.make_async_copy(rows_s[b], acc.at[dst_v.at[g]],
                                  ssem[b]).wait()

        start_gather(0, 0)
        start_gather(1, 1)

        M_HI = jnp.int32(-65536)

        def outer(t, carry):
            for b in range(NB):
                g = t * NB + b
                wait_gather(b, g)
                bp = (b + 2) % NB
                @pl.when(g + 2 < CPT)
                def _next():
                    start_gather(bp, g + 2)
                @pl.when(t > 0)
                def _drain():
                    wait_scatter(b, g - NB)
                for i in range(C // 16):
                    wch[b][pl.ds(16 * i, 16)] = (
                        wch[b][pl.ds(16 * i, 16)] * av16)
                def scale(eg, c2):
                    wv16 = wch[b][pl.ds(16 * eg, 16)]
                    for l in range(16):
                        e = 16 * eg + l
                        for j in range(HD // 32):
                            v = plsc.bitcast(
                                rows_g[b][e, pl.ds(32 * j, 32)], jnp.int32)
                            lo = plsc.bitcast(jnp.left_shift(v, 16),
                                              jnp.float32)
                            hi = plsc.bitcast(jnp.bitwise_and(v, M_HI),
                                              jnp.float32)
                            rows_s[b][e, pl.ds(32 * j, 16)] = lo * wv16[l]
                            rows_s[b][e, pl.ds(32 * j + 16, 16)] = (
                                hi * wv16[l])
                    return c2
                lax.fori_loop(0, C // 16, scale, 0)
                start_scatter(b, g)
            return carry
        lax.fori_loop(0, CPT // NB, outer, 0)
        for b in range(NB):
            wait_scatter(b, CPT - NB + b)
        plsc.subcore_barrier()

        for off in range(0, RPS, C):
            sz = min(C, RPS - off)
            pltpu.sync_copy(acc.at[pl.ds(sid * RPS + off, sz)],
                            rows_s[0].at[pl.ds(0, sz)])
            pltpu.sync_copy(rows_s[0].at[pl.ds(0, sz)],
                            out_hbm.at[cid].at[pl.ds(sid * RPS + off, sz)])
        @pl.when(sid == NS - 1)
        def _write_tail():
            pltpu.sync_copy(acc.at[pl.ds(TAIL0, TAILN)],
                            rows_s[1].at[pl.ds(0, TAILN)])
            pltpu.sync_copy(rows_s[1].at[pl.ds(0, TAILN)],
                            out_hbm.at[cid].at[pl.ds(TAIL0, TAILN)])

    return k(z3, src2, dst2, w1, a8)


def kernel(Z, edge_index, edge_weight, alpha):
    src = edge_index[0].astype(jnp.int32)
    dst = edge_index[1].astype(jnp.int32)
    w = edge_weight.astype(jnp.float32)
    pad = EPAD - src.shape[0]
    src2 = jnp.pad(src, (0, pad)).reshape(EPAD // C, C)
    dst2 = jnp.pad(dst, (0, pad)).reshape(EPAD // C, C)
    w1 = jnp.pad(w, (0, pad)).reshape(EPAD // C, C)
    a16 = jnp.broadcast_to(alpha.astype(jnp.float32)[None], (16,))
    zb = Z.astype(jnp.bfloat16)
    z_t = jnp.swapaxes(zb.reshape(N, NC, HD), 0, 1)
    out_t = _sc_spmm(z_t, src2, dst2, w1, a16)
    out4 = out_t.reshape(NC, N, HD // 32, 2, 16)
    out5 = jnp.transpose(out4, (0, 1, 2, 4, 3)).reshape(NC, N, HD)
    return jnp.swapaxes(out5, 0, 1).reshape(N, D)


# speedup vs baseline: 2.5951x; 1.3655x over previous
import functools

import jax
import jax.numpy as jnp
from jax import lax
from jax.experimental import pallas as pl
from jax.experimental.pallas import tpu as pltpu
from jax.experimental.pallas import tpu_sc as plsc

N = 10000
D = 128
HD = D // 2
NC = 2
NS = 16
C = 128
CPT = 159
EPT = C * CPT
EPAD = EPT * NS
NB = 3
RPS = 624
TAIL0 = RPS * NS
TAILN = N - TAIL0


def _sc_spmm(z3, src2, dst2, w1, a8):
    mesh = plsc.VectorSubcoreMesh(core_axis_name="c", subcore_axis_name="s",
                                  num_cores=NC, num_subcores=NS)

    @functools.partial(
        pl.kernel,
        out_type=jax.ShapeDtypeStruct((NC, N, HD), jnp.float32),
        mesh=mesh,
        compiler_params=pltpu.CompilerParams(use_tc_tiling_on_sc=False,
                                             needs_layout_passes=False),
        scratch_types=dict(
            src_v=pltpu.VMEM((CPT, C), jnp.int32),
            dst_v=pltpu.VMEM((CPT, C), jnp.int32),
            wch=[pltpu.VMEM((C,), jnp.float32) for _ in range(NB)],
            a_v=pltpu.VMEM((16,), jnp.float32),
            rows_g=[pltpu.VMEM((C, HD), jnp.bfloat16) for _ in range(NB)],
            rows_s=[pltpu.VMEM((C, HD), jnp.float32) for _ in range(NB)],
            gsem=[pltpu.SemaphoreType.DMA for _ in range(NB)],
            ssem=[pltpu.SemaphoreType.DMA for _ in range(NB)],
            wsem=[pltpu.SemaphoreType.DMA for _ in range(NB)],
            acc=pltpu.VMEM_SHARED((N, HD), jnp.float32),
        ),
    )
    def k(z_hbm, src_hbm, dst_hbm, w_hbm, a_hbm, out_hbm,
          src_v, dst_v, wch, a_v, rows_g, rows_s, gsem, ssem, wsem, acc):
        cid = lax.axis_index("c")
        sid = lax.axis_index("s")

        def zrow(e, carry):
            for j in range(HD // 16):
                rows_s[0][e, pl.ds(16 * j, 16)] = jnp.zeros((16,), jnp.float32)
            return carry
        lax.fori_loop(0, C, zrow, 0)
        for off in range(0, RPS, C):
            sz = min(C, RPS - off)
            pltpu.sync_copy(rows_s[0].at[pl.ds(0, sz)],
                            acc.at[pl.ds(sid * RPS + off, sz)])
        @pl.when(sid == NS - 1)
        def _zero_tail():
            pltpu.sync_copy(rows_s[0].at[pl.ds(0, TAILN)],
                            acc.at[pl.ds(TAIL0, TAILN)])
        plsc.subcore_barrier()

        pltpu.sync_copy(src_hbm.at[pl.ds(sid * CPT, CPT)], src_v)
        pltpu.sync_copy(dst_hbm.at[pl.ds(sid * CPT, CPT)], dst_v)
        pltpu.sync_copy(a_hbm, a_v)
        av16 = a_v[...]

        def start_gather(b, g):
            pltpu.async_copy(w_hbm.at[sid * CPT + g], wch[b], wsem[b])
            pltpu.async_copy(z_hbm.at[cid].at[src_v.at[g]], rows_g[b],
                             gsem[b])

        def wait_gather(b, g):
            pltpu.make_async_copy(w_hbm.at[sid * CPT + g], wch[b],
                                  wsem[b]).wait()
            pltpu.make_async_copy(z_hbm.at[cid].at[src_v.at[g]], rows_g[b],
                                  gsem[b]).wait()

        def start_scatter(b, g):
            pltpu.async_copy(rows_s[b], acc.at[dst_v.at[g]], ssem[b],
                             add=True)

        def wait_scatter(b, g):
            pltpu.make_async_copy(rows_s[b], acc.at[dst_v.at[g]],
                                  ssem[b]).wait()

        start_gather(0, 0)
        start_gather(1, 1)

        M_HI = jnp.int32(-65536)

        def outer(t, carry):
            for b in range(NB):
                g = t * NB + b
                wait_gather(b, g)
                bp = (b + 2) % NB
                @pl.when(g + 2 < CPT)
                def _next():
                    start_gather(bp, g + 2)
                @pl.when(t > 0)
                def _drain():
                    wait_scatter(b, g - NB)
                for i in range(C // 16):
                    wch[b][pl.ds(16 * i, 16)] = (
                        wch[b][pl.ds(16 * i, 16)] * av16)
                @plsc.parallel_loop(0, C, step=16)
                def scale(e0):
                    wv16 = wch[b][pl.ds(e0, 16)]
                    for l in range(16):
                        e = e0 + l
                        for j in range(HD // 32):
                            v = plsc.bitcast(
                                rows_g[b][e, pl.ds(32 * j, 32)], jnp.int32)
                            lo = plsc.bitcast(jnp.left_shift(v, 16),
                                              jnp.float32)
                            hi = plsc.bitcast(jnp.bitwise_and(v, M_HI),
                                              jnp.float32)
                            rows_s[b][e, pl.ds(32 * j, 16)] = lo * wv16[l]
                            rows_s[b][e, pl.ds(32 * j + 16, 16)] = (
                                hi * wv16[l])
                start_scatter(b, g)
            return carry
        lax.fori_loop(0, CPT // NB, outer, 0)
        for b in range(NB):
            wait_scatter(b, CPT - NB + b)
        plsc.subcore_barrier()

        for off in range(0, RPS, C):
            sz = min(C, RPS - off)
            pltpu.sync_copy(acc.at[pl.ds(sid * RPS + off, sz)],
                            rows_s[0].at[pl.ds(0, sz)])
            pltpu.sync_copy(rows_s[0].at[pl.ds(0, sz)],
                            out_hbm.at[cid].at[pl.ds(sid * RPS + off, sz)])
        @pl.when(sid == NS - 1)
        def _write_tail():
            pltpu.sync_copy(acc.at[pl.ds(TAIL0, TAILN)],
                            rows_s[1].at[pl.ds(0, TAILN)])
            pltpu.sync_copy(rows_s[1].at[pl.ds(0, TAILN)],
                            out_hbm.at[cid].at[pl.ds(TAIL0, TAILN)])

    return k(z3, src2, dst2, w1, a8)


def kernel(Z, edge_index, edge_weight, alpha):
    src = edge_index[0].astype(jnp.int32)
    dst = edge_index[1].astype(jnp.int32)
    w = edge_weight.astype(jnp.float32)
    pad = EPAD - src.shape[0]
    src2 = jnp.pad(src, (0, pad)).reshape(EPAD // C, C)
    dst2 = jnp.pad(dst, (0, pad)).reshape(EPAD // C, C)
    w1 = jnp.pad(w, (0, pad)).reshape(EPAD // C, C)
    a16 = jnp.broadcast_to(alpha.astype(jnp.float32)[None], (16,))
    zb = Z.astype(jnp.bfloat16)
    z_t = jnp.swapaxes(zb.reshape(N, NC, HD), 0, 1)
    out_t = _sc_spmm(z_t, src2, dst2, w1, a16)
    out4 = out_t.reshape(NC, N, HD // 32, 2, 16)
    out5 = jnp.transpose(out4, (0, 1, 2, 4, 3)).reshape(NC, N, HD)
    return jnp.swapaxes(out5, 0, 1).reshape(N, D)
